# Initial kernel scaffold; baseline (speedup 1.0000x reference)
import jax
import jax.numpy as jnp
from jax.experimental import pallas as pl


def kernel(token_embeddings, modality_ids, pos_tables):
    raise NotImplementedError("write your pallas kernel here")


# speedup vs baseline: 1.5776x; 1.5776x over previous
import functools

import jax
import jax.numpy as jnp
from jax import lax
from jax.experimental import pallas as pl
from jax.experimental.pallas import tpu as pltpu
from jax.experimental.pallas import tpu_sc as plsc

B = 4
T = 4096
D = 768
N_MOD = 4
MAX_SEQ = 4096

NC = 2
NS = 16
NW = NC * NS
ROW_W = NW // B
CHUNK = T // ROW_W
G = 64
NG = CHUNK // G
DV = D // 16


def _body(tok_hbm, ids_hbm, pe_hbm, out_hbm, ids_v, idx_v, tok_v, pe_v, sem):
    cid = lax.axis_index("c")
    sid = lax.axis_index("s")
    wid = cid * NS + sid
    b = wid // ROW_W
    k = wid % ROW_W

    pltpu.sync_copy(ids_hbm.at[pl.ds(b * T, T)], ids_v)

    zeros = jnp.zeros((16,), jnp.int32)
    ones = jnp.ones((16,), jnp.int32)
    mvecs = [jnp.full((16,), m, jnp.int32) for m in range(N_MOD)]

    def pre_body(i, cnts):
        v = ids_v[pl.ds(i * 16, 16)]
        return tuple(
            cnts[m] + jnp.sum(jnp.where(v == mvecs[m], ones, zeros))
            for m in range(N_MOD)
        )

    cnts = lax.fori_loop(0, k * (CHUNK // 16), pre_body, (0, 0, 0, 0))

    base = k * CHUNK

    def rank_body(i, cnts):
        v = ids_v[pl.ds(base + i * 16, 16)]
        idx = v * jnp.full((16,), MAX_SEQ, jnp.int32)
        new = []
        for m in range(N_MOD):
            mk = jnp.where(v == mvecs[m], ones, zeros)
            pre = plsc.cumsum(mk)
            cnt_b = jnp.full((16,), cnts[m], jnp.int32)
            idx = idx + mk * (cnt_b + pre - ones)
            new.append(cnts[m] + jnp.sum(mk))
        idx_v[pl.ds(i * 16, 16)] = idx
        return tuple(new)

    lax.fori_loop(0, CHUNK // 16, rank_body, cnts)

    gbase = wid * CHUNK

    def tile_body(t, _):
        off = (gbase + t * G) * D
        pltpu.sync_copy(tok_hbm.at[pl.ds(off, G * D)], tok_v)
        pltpu.async_copy(pe_hbm.at[idx_v.at[pl.ds(t * G, G)]], pe_v, sem).wait()

        def add_row(r, _):
            for j in range(DV):
                sl = pl.ds(r * D + j * 16, 16)
                tok_v[sl] = tok_v[sl] + pe_v[r, pl.ds(j * 16, 16)]
            return 0

        lax.fori_loop(0, G, add_row, 0)
        pltpu.sync_copy(tok_v, out_hbm.at[pl.ds(off, G * D)])
        return 0

    lax.fori_loop(0, NG, tile_body, 0)


@jax.jit
def kernel(token_embeddings, modality_ids, pos_tables):
    mesh = plsc.VectorSubcoreMesh(
        core_axis_name="c", subcore_axis_name="s", num_cores=NC, num_subcores=NS
    )
    kern = functools.partial(
        pl.kernel,
        mesh=mesh,
        compiler_params=pltpu.CompilerParams(needs_layout_passes=False),
        out_type=jax.ShapeDtypeStruct((B * T * D,), jnp.float32),
        scratch_types=[
            pltpu.VMEM((T,), jnp.int32),
            pltpu.VMEM((CHUNK,), jnp.int32),
            pltpu.VMEM((G * D,), jnp.float32),
            pltpu.VMEM((G, D), jnp.float32),
            pltpu.SemaphoreType.DMA,
        ],
    )(_body)
    out = kern(
        token_embeddings.reshape(B * T * D),
        modality_ids.reshape(B * T),
        pos_tables.reshape(N_MOD * MAX_SEQ, D),
    )
    return out.reshape(B, T, D)


# speedup vs baseline: 1.7294x; 1.0962x over previous
import functools

import jax
import jax.numpy as jnp
from jax import lax
from jax.experimental import pallas as pl
from jax.experimental.pallas import tpu as pltpu
from jax.experimental.pallas import tpu_sc as plsc

B = 4
T = 4096
D = 768
N_MOD = 4
MAX_SEQ = 4096

NC = 2
NS = 16
NW = NC * NS
ROW_W = NW // B
CHUNK = T // ROW_W
G = 32
NT = CHUNK // G
DV = D // 16


def _body(tok_hbm, ids_hbm, pe_hbm, out_hbm,
          ids_v, idx_v, tok_v, pe_v, tok_sem, pe_sem, out_sem):
    cid = lax.axis_index("c")
    sid = lax.axis_index("s")
    wid = cid * NS + sid
    b = wid // ROW_W
    k = wid % ROW_W

    pltpu.sync_copy(ids_hbm.at[pl.ds(b * T, T)], ids_v)

    zeros = jnp.zeros((16,), jnp.int32)
    ones = jnp.ones((16,), jnp.int32)
    mvecs = [jnp.full((16,), m, jnp.int32) for m in range(N_MOD)]

    def pre_body(i, cnts):
        v = ids_v[pl.ds(i * 16, 16)]
        return tuple(
            cnts[m] + jnp.sum(jnp.where(v == mvecs[m], ones, zeros))
            for m in range(N_MOD)
        )

    cnts = lax.fori_loop(0, k * (CHUNK // 16), pre_body, (0, 0, 0, 0))

    base = k * CHUNK

    def rank_body(i, cnts):
        v = ids_v[pl.ds(base + i * 16, 16)]
        idx = v * jnp.full((16,), MAX_SEQ, jnp.int32)
        new = []
        for m in range(N_MOD):
            mk = jnp.where(v == mvecs[m], ones, zeros)
            pre = plsc.cumsum(mk)
            cnt_b = jnp.full((16,), cnts[m], jnp.int32)
            idx = idx + mk * (cnt_b + pre - ones)
            new.append(cnts[m] + jnp.sum(mk))
        idx_v[pl.ds(i * 16, 16)] = idx
        return tuple(new)

    lax.fori_loop(0, CHUNK // 16, rank_body, cnts)

    gbase = wid * CHUNK

    def in_tok(t, s):
        off = (gbase + t * G) * D
        return pltpu.make_async_copy(
            tok_hbm.at[pl.ds(off, G * D)], tok_v.at[s], tok_sem)

    def in_pe(t, s):
        return pltpu.make_async_copy(
            pe_hbm.at[idx_v.at[pl.ds(t * G, G)]], pe_v.at[s], pe_sem)

    def out_cp(t, s):
        off = (gbase + t * G) * D
        return pltpu.make_async_copy(
            tok_v.at[s], out_hbm.at[pl.ds(off, G * D)], out_sem)

    in_tok(0, 0).start()
    in_pe(0, 0).start()

    def tile_body(t, _):
        s = lax.rem(t, 2)
        sn = lax.rem(t + 1, 2)

        @pl.when(t + 1 < NT)
        def _():
            @pl.when(t >= 1)
            def _():
                out_cp(t - 1, sn).wait()

            in_tok(t + 1, sn).start()
            in_pe(t + 1, sn).start()

        in_tok(t, s).wait()
        in_pe(t, s).wait()

        def add_row(r, _):
            for j in range(DV):
                sl = pl.ds(r * D + j * 16, 16)
                tok_v[s, sl] = tok_v[s, sl] + pe_v[s, r, pl.ds(j * 16, 16)]
            return 0

        lax.fori_loop(0, G, add_row, 0)
        out_cp(t, s).start()
        return 0

    lax.fori_loop(0, NT, tile_body, 0)
    out_cp(NT - 2, lax.rem(NT - 2, 2)).wait()
    out_cp(NT - 1, lax.rem(NT - 1, 2)).wait()


@jax.jit
def kernel(token_embeddings, modality_ids, pos_tables):
    mesh = plsc.VectorSubcoreMesh(
        core_axis_name="c", subcore_axis_name="s", num_cores=NC, num_subcores=NS
    )
    kern = functools.partial(
        pl.kernel,
        mesh=mesh,
        compiler_params=pltpu.CompilerParams(needs_layout_passes=False),
        out_type=jax.ShapeDtypeStruct((B * T * D,), jnp.float32),
        scratch_types=[
            pltpu.VMEM((T,), jnp.int32),
            pltpu.VMEM((CHUNK,), jnp.int32),
            pltpu.VMEM((2, G * D), jnp.float32),
            pltpu.VMEM((2, G, D), jnp.float32),
            pltpu.SemaphoreType.DMA,
            pltpu.SemaphoreType.DMA,
            pltpu.SemaphoreType.DMA,
        ],
    )(_body)
    out = kern(
        token_embeddings.reshape(B * T * D),
        modality_ids.reshape(B * T),
        pos_tables.reshape(N_MOD * MAX_SEQ, D),
    )
    return out.reshape(B, T, D)


# speedup vs baseline: 6.1303x; 3.5448x over previous
import functools

import jax
import jax.numpy as jnp
from jax import lax
from jax.experimental import pallas as pl
from jax.experimental.pallas import tpu as pltpu
from jax.experimental.pallas import tpu_sc as plsc

B = 4
T = 4096
D = 768
N_MOD = 4
MAX_SEQ = 4096

NC = 2
NS = 16
NW = NC * NS
ROW_W = NW // B
CHUNK = T // ROW_W
G = 32
NT = CHUNK // G
DV = D // 16


def _body(tok_hbm, ids_hbm, pe_hbm, out_hbm,
          ids_v, idx_v, tok_v, pe_v, tok_sem, pe_sem, out_sem):
    cid = lax.axis_index("c")
    sid = lax.axis_index("s")
    wid = cid * NS + sid
    b = wid // ROW_W
    k = wid % ROW_W

    pltpu.sync_copy(ids_hbm.at[b], ids_v)

    zeros = jnp.zeros((16,), jnp.int32)
    ones = jnp.ones((16,), jnp.int32)
    mvecs = [jnp.full((16,), m, jnp.int32) for m in range(N_MOD)]

    def pre_body(i, cnts):
        v = ids_v[pl.ds(i * 16, 16)]
        return tuple(
            cnts[m] + jnp.sum(jnp.where(v == mvecs[m], ones, zeros))
            for m in range(N_MOD)
        )

    cnts = lax.fori_loop(0, k * (CHUNK // 16), pre_body, (0, 0, 0, 0))

    base = k * CHUNK

    def rank_body(i, cnts):
        v = ids_v[pl.ds(base + i * 16, 16)]
        idx = v * jnp.full((16,), MAX_SEQ, jnp.int32)
        new = []
        for m in range(N_MOD):
            mk = jnp.where(v == mvecs[m], ones, zeros)
            pre = plsc.cumsum(mk)
            cnt_b = jnp.full((16,), cnts[m], jnp.int32)
            idx = idx + mk * (cnt_b + pre - ones)
            new.append(cnts[m] + jnp.sum(mk))
        idx_v[pl.ds(i * 16, 16)] = idx
        return tuple(new)

    lax.fori_loop(0, CHUNK // 16, rank_body, cnts)

    gbase = wid * CHUNK

    def in_tok(t, s):
        row = gbase + t * G
        return pltpu.make_async_copy(
            tok_hbm.at[pl.ds(row, G)], tok_v.at[s], tok_sem)

    def in_pe(t, s):
        return pltpu.make_async_copy(
            pe_hbm.at[idx_v.at[pl.ds(t * G, G)]], pe_v.at[s], pe_sem)

    def out_cp(t, s):
        row = gbase + t * G
        return pltpu.make_async_copy(
            tok_v.at[s], out_hbm.at[pl.ds(row, G)], out_sem)

    in_tok(0, 0).start()
    in_pe(0, 0).start()

    def tile_body(t, _):
        s = lax.rem(t, 2)
        sn = lax.rem(t + 1, 2)

        @pl.when(t + 1 < NT)
        def _():
            @pl.when(t >= 1)
            def _():
                out_cp(t - 1, sn).wait()

            in_tok(t + 1, sn).start()
            in_pe(t + 1, sn).start()

        in_tok(t, s).wait()
        in_pe(t, s).wait()

        def add_row(r, _):
            for j in range(DV):
                sl = pl.ds(j * 16, 16)
                tok_v[s, r, sl] = tok_v[s, r, sl] + pe_v[s, r, sl]
            return 0

        lax.fori_loop(0, G, add_row, 0)
        out_cp(t, s).start()
        return 0

    lax.fori_loop(0, NT, tile_body, 0)
    out_cp(NT - 2, lax.rem(NT - 2, 2)).wait()
    out_cp(NT - 1, lax.rem(NT - 1, 2)).wait()


@jax.jit
def kernel(token_embeddings, modality_ids, pos_tables):
    mesh = plsc.VectorSubcoreMesh(
        core_axis_name="c", subcore_axis_name="s", num_cores=NC, num_subcores=NS
    )
    kern = functools.partial(
        pl.kernel,
        mesh=mesh,
        compiler_params=pltpu.CompilerParams(needs_layout_passes=False),
        out_type=jax.ShapeDtypeStruct((B * T, D), jnp.float32),
        scratch_types=[
            pltpu.VMEM((T,), jnp.int32),
            pltpu.VMEM((CHUNK,), jnp.int32),
            pltpu.VMEM((2, G, D), jnp.float32),
            pltpu.VMEM((2, G, D), jnp.float32),
            pltpu.SemaphoreType.DMA,
            pltpu.SemaphoreType.DMA,
            pltpu.SemaphoreType.DMA,
        ],
    )(_body)
    out = kern(
        token_embeddings.reshape(B * T, D),
        modality_ids,
        pos_tables.reshape(N_MOD * MAX_SEQ, D),
    )
    return out.reshape(B, T, D)
